# Initial kernel scaffold; baseline (speedup 1.0000x reference)
import jax
import jax.numpy as jnp
from jax.experimental import pallas as pl


def kernel(x, edge_index, edge_weight, W, b):
    raise NotImplementedError("write your pallas kernel here")


# speedup vs baseline: 4.3142x; 4.3142x over previous
import functools

import jax
import jax.numpy as jnp
from jax import lax
from jax.experimental import pallas as pl
from jax.experimental.pallas import tpu as pltpu
from jax.experimental.pallas import tpu_sc as plsc

N = 10000
E = 320000
D = 128

NC = 2
NS = 16
NW = NC * NS
EPT = E // NW
K = 80
CPT = EPT // K
RB = 80
NCHUNK = N // RB
LANES = 8


def _mm_body(x_ref, w_ref, o_ref):
    o_ref[...] = jnp.dot(x_ref[...], w_ref[...],
                         preferred_element_type=jnp.float32)


def _combine_body(p_ref, b_ref, o_ref):
    o_ref[...] = p_ref[0] + p_ref[1] + b_ref[...]


def _sc_scatter(support, edges, wts):
    mesh = plsc.VectorSubcoreMesh(core_axis_name="c", subcore_axis_name="s")

    @functools.partial(
        pl.kernel,
        mesh=mesh,
        out_type=jax.ShapeDtypeStruct((NC, N, D), jnp.float32),
        scratch_types=[
            pltpu.VMEM((2, K), jnp.int32),
            pltpu.VMEM((1, K), jnp.float32),
            pltpu.VMEM((K, D), jnp.float32),
            pltpu.VMEM_SHARED((N, D), jnp.float32),
            pltpu.SemaphoreType.DMA,
        ],
    )
    def scatter_kernel(support_hbm, edges_hbm, wts_hbm, out_hbm,
                       ebuf, wbuf, rows, acc, sem):
        c = lax.axis_index("c")
        s = lax.axis_index("s")
        wid = c * NS + s

        zeros16 = jnp.zeros((16,), jnp.float32)

        @pl.loop(0, RB)
        def _(i):
            for j in range(LANES):
                rows[i, pl.ds(j * 16, 16)] = zeros16

        for i in range((NCHUNK + NS - 1) // NS):
            ck = s + i * NS

            @pl.when(ck < NCHUNK)
            def _():
                pltpu.sync_copy(rows, acc.at[pl.ds(ck * RB, RB)])
        plsc.subcore_barrier()

        @pl.loop(0, CPT)
        def _(g):
            pltpu.sync_copy(edges_hbm.at[wid, g], ebuf)
            pltpu.sync_copy(wts_hbm.at[wid, g], wbuf)
            pltpu.async_copy(support_hbm.at[ebuf.at[1]], rows, sem).wait()

            @pl.loop(0, K // 16)
            def _(eg):
                wvec = wbuf[0, pl.ds(eg * 16, 16)]
                for l in range(16):
                    wl = jnp.broadcast_to(wvec[l], (16,))
                    e = eg * 16 + l
                    for j in range(LANES):
                        sl = pl.ds(j * 16, 16)
                        rows[e, sl] = rows[e, sl] * wl

            pltpu.sync_copy(rows, acc.at[ebuf.at[0]], add=True)

        plsc.subcore_barrier()

        for i in range((NCHUNK + NS - 1) // NS):
            ck = s + i * NS

            @pl.when(ck < NCHUNK)
            def _():
                pltpu.sync_copy(acc.at[pl.ds(ck * RB, RB)], rows)
                pltpu.sync_copy(rows, out_hbm.at[c, pl.ds(ck * RB, RB)])

    return scatter_kernel(support, edges, wts)


def kernel(x, edge_index, edge_weight, W, b):
    support = pl.pallas_call(
        _mm_body,
        grid=(25,),
        in_specs=[
            pl.BlockSpec((400, D), lambda i: (i, 0)),
            pl.BlockSpec((D, D), lambda i: (0, 0)),
        ],
        out_specs=pl.BlockSpec((400, D), lambda i: (i, 0)),
        out_shape=jax.ShapeDtypeStruct((N, D), jnp.float32),
    )(x, W)

    edges = edge_index.reshape(2, NW, CPT, K).transpose(1, 2, 0, 3)
    wts = edge_weight.reshape(1, NW, CPT, K).transpose(1, 2, 0, 3)

    partial = _sc_scatter(support, edges, wts)

    out = pl.pallas_call(
        _combine_body,
        grid=(25,),
        in_specs=[
            pl.BlockSpec((NC, 400, D), lambda i: (0, i, 0)),
            pl.BlockSpec((1, D), lambda i: (0, 0)),
        ],
        out_specs=pl.BlockSpec((400, D), lambda i: (i, 0)),
        out_shape=jax.ShapeDtypeStruct((N, D), jnp.float32),
    )(partial, b.reshape(1, D))
    return out


# speedup vs baseline: 8.4159x; 1.9508x over previous
import functools

import jax
import jax.numpy as jnp
from jax import lax
from jax.experimental import pallas as pl
from jax.experimental.pallas import tpu as pltpu
from jax.experimental.pallas import tpu_sc as plsc

N = 10000
E = 320000
D = 128

NC = 2
NS = 16
NW = NC * NS
EPT = E // NW
K = 80
CPT = EPT // K
RB = 80
NCHUNK = N // RB
LANES = 8


def _mm_body(x_ref, w_ref, o_ref):
    o_ref[...] = jnp.dot(x_ref[...], w_ref[...],
                         preferred_element_type=jnp.float32)


def _combine_body(p_ref, b_ref, o_ref):
    o_ref[...] = p_ref[0] + p_ref[1] + b_ref[...]


def _sc_scatter(support, edges, wts):
    mesh = plsc.VectorSubcoreMesh(core_axis_name="c", subcore_axis_name="s")

    @functools.partial(
        pl.kernel,
        mesh=mesh,
        out_type=jax.ShapeDtypeStruct((NC, N, D), jnp.float32),
        scratch_types=[
            pltpu.VMEM((2, K), jnp.int32),
            pltpu.VMEM((2, K), jnp.int32),
            pltpu.VMEM((1, K), jnp.float32),
            pltpu.VMEM((1, K), jnp.float32),
            pltpu.VMEM((1, K), jnp.int32),
            pltpu.VMEM((1, K), jnp.int32),
            pltpu.VMEM((K, D), jnp.float32),
            pltpu.VMEM((K, D), jnp.float32),
            pltpu.VMEM_SHARED((N, D), jnp.float32),
            pltpu.SemaphoreType.DMA,
            pltpu.SemaphoreType.DMA,
            pltpu.SemaphoreType.DMA,
        ],
    )
    def scatter_kernel(support_hbm, edges_hbm, wts_hbm, out_hbm,
                       ebuf0, ebuf1, wbuf0, wbuf1, dbuf0, dbuf1,
                       rows0, rows1, acc, esem, gsem, ssem):
        c = lax.axis_index("c")
        s = lax.axis_index("s")
        wid = c * NS + s

        ebuf = (ebuf0, ebuf1)
        wbuf = (wbuf0, wbuf1)
        dbuf = (dbuf0, dbuf1)
        rows = (rows0, rows1)

        zeros16 = jnp.zeros((16,), jnp.float32)

        @pl.loop(0, RB)
        def _(i):
            for j in range(LANES):
                rows0[i, pl.ds(j * 16, 16)] = zeros16

        for i in range((NCHUNK + NS - 1) // NS):
            ck = s + i * NS

            @pl.when(ck < NCHUNK)
            def _():
                pltpu.sync_copy(rows0, acc.at[pl.ds(ck * RB, RB)])
        plsc.subcore_barrier()

        def multiply(p):
            @pl.loop(0, K // 16)
            def _(eg):
                wvec = wbuf[p][0, pl.ds(eg * 16, 16)]
                for l in range(16):
                    wl = jnp.broadcast_to(wvec[l], (16,))
                    e = eg * 16 + l
                    for j in range(LANES):
                        sl = pl.ds(j * 16, 16)
                        rows[p][e, sl] = rows[p][e, sl] * wl
            for i in range(K // 16):
                sl = pl.ds(i * 16, 16)
                dbuf[p][0, sl] = ebuf[p][0, sl]

        def step(g, p):
            pltpu.make_async_copy(
                support_hbm.at[ebuf[p].at[1]], rows[p], gsem).wait()

            @pl.when(g + 1 < CPT)
            def _():
                pltpu.make_async_copy(
                    edges_hbm.at[wid, g + 1], ebuf[1 - p], esem).wait()
                pltpu.make_async_copy(
                    wts_hbm.at[wid, g + 1], wbuf[1 - p], esem).wait()

            @pl.when(g > 0)
            def _():
                pltpu.make_async_copy(
                    rows[1 - p], acc.at[dbuf[1 - p].at[0]], ssem).wait()

            @pl.when(g + 1 < CPT)
            def _():
                pltpu.async_copy(
                    support_hbm.at[ebuf[1 - p].at[1]], rows[1 - p], gsem)

            multiply(p)

            @pl.when(g + 2 < CPT)
            def _():
                pltpu.async_copy(edges_hbm.at[wid, g + 2], ebuf[p], esem)
                pltpu.async_copy(wts_hbm.at[wid, g + 2], wbuf[p], esem)

            pltpu.async_copy(rows[p], acc.at[dbuf[p].at[0]], ssem, add=True)

        pltpu.sync_copy(edges_hbm.at[wid, 0], ebuf0)
        pltpu.sync_copy(wts_hbm.at[wid, 0], wbuf0)
        pltpu.async_copy(support_hbm.at[ebuf0.at[1]], rows0, gsem)
        pltpu.async_copy(edges_hbm.at[wid, 1], ebuf1, esem)
        pltpu.async_copy(wts_hbm.at[wid, 1], wbuf1, esem)

        @pl.loop(0, CPT, step=2)
        def _(g):
            step(g, 0)

            @pl.when(g + 1 < CPT)
            def _():
                step(g + 1, 1)

        pltpu.make_async_copy(rows0, acc.at[dbuf0.at[0]], ssem).wait()

        plsc.subcore_barrier()

        for i in range((NCHUNK + NS - 1) // NS):
            ck = s + i * NS

            @pl.when(ck < NCHUNK)
            def _():
                pltpu.sync_copy(acc.at[pl.ds(ck * RB, RB)], rows0)
                pltpu.sync_copy(rows0, out_hbm.at[c, pl.ds(ck * RB, RB)])

    return scatter_kernel(support, edges, wts)


def kernel(x, edge_index, edge_weight, W, b):
    support = pl.pallas_call(
        _mm_body,
        grid=(25,),
        in_specs=[
            pl.BlockSpec((400, D), lambda i: (i, 0)),
            pl.BlockSpec((D, D), lambda i: (0, 0)),
        ],
        out_specs=pl.BlockSpec((400, D), lambda i: (i, 0)),
        out_shape=jax.ShapeDtypeStruct((N, D), jnp.float32),
    )(x, W)

    edges = edge_index.reshape(2, NW, CPT, K).transpose(1, 2, 0, 3)
    wts = edge_weight.reshape(1, NW, CPT, K).transpose(1, 2, 0, 3)

    partial = _sc_scatter(support, edges, wts)

    out = pl.pallas_call(
        _combine_body,
        grid=(25,),
        in_specs=[
            pl.BlockSpec((NC, 400, D), lambda i: (0, i, 0)),
            pl.BlockSpec((1, D), lambda i: (0, 0)),
        ],
        out_specs=pl.BlockSpec((400, D), lambda i: (i, 0)),
        out_shape=jax.ShapeDtypeStruct((N, D), jnp.float32),
    )(partial, b.reshape(1, D))
    return out
